# baseline (speedup 1.0000x reference)
import functools

import jax
import jax.numpy as jnp
from jax import lax
from jax.experimental import pallas as pl
from jax.experimental.pallas import tpu as pltpu
from jax.experimental.pallas import tpu_sc as plsc

_INFO = plsc.get_sparse_core_info()
_NC, _NS, _LANES = _INFO.num_cores, _INFO.num_subcores, _INFO.num_lanes
_NW = _NC * _NS
_BB = 128
_NBUF = 4


def _build(B, L, E):
    assert E % 8 == 0 and B % _BB == 0
    eh = E // 8
    nbh = B // _BB
    units = L * nbh
    assert units % _NW == 0
    per_w = units // _NW
    assert per_w % _NBUF == 0

    mesh = plsc.VectorSubcoreMesh(core_axis_name="c", subcore_axis_name="s")

    @functools.partial(
        pl.kernel,
        mesh=mesh,
        out_type=jax.ShapeDtypeStruct((L, eh, nbh, 8, _BB), jnp.float32),
        compiler_params=pltpu.CompilerParams(use_tc_tiling_on_sc=False,
                                             needs_layout_passes=False),
        scratch_types=(
            [
                pltpu.VMEM((per_w, _BB), jnp.int32),
                pltpu.VMEM((L, E), jnp.float32),
                pltpu.VMEM((_NBUF, _BB, E), jnp.float32),
                pltpu.VMEM((_NBUF, eh, 8, _BB), jnp.float32),
            ]
            + [pltpu.SemaphoreType.DMA] * (2 * _NBUF)
        ),
    )
    def run(srcT_hbm, word_hbm, pos_hbm, out_hbm, idx_v, pos_v, rows_v,
            cbuf_v, *sems):
        gsems, osems = sems[:_NBUF], sems[_NBUF:]
        wid = lax.axis_index("s") * _NC + lax.axis_index("c")
        u0 = wid * per_w
        pltpu.sync_copy(srcT_hbm.at[pl.ds(u0, per_w)], idx_v)
        pltpu.sync_copy(pos_hbm.at[pl.ds(0, L)], pos_v)

        blk_rows = [lax.iota(jnp.int32, 16) + (16 * k) for k in range(8)]

        def issue_gather(i, b):
            pltpu.async_copy(word_hbm.at[idx_v.at[i]], rows_v.at[b], gsems[b])

        def wait_gather(b):
            pltpu.make_async_copy(word_hbm.at[idx_v.at[0]], rows_v.at[b],
                                  gsems[b]).wait()

        def wait_out(b):
            pltpu.make_async_copy(cbuf_v.at[b],
                                  out_hbm.at[0, pl.ds(0, eh), 0], osems[b]).wait()

        issue_gather(0, 0)
        issue_gather(1, 1)

        def group_body(g, carry):
            for b in range(_NBUF):
                i = g * _NBUF + b
                inext = i + 2
                bnext = (b + 2) % _NBUF

                @pl.when(inext < per_w)
                def _prefetch():
                    issue_gather(inext, bnext)

                wait_gather(b)

                @pl.when(i >= _NBUF)
                def _drain():
                    wait_out(b)

                u = u0 + i
                lpos = u >> 5
                bh = u & (nbh - 1)

                lv = jnp.full((16,), lpos, jnp.int32)

                def e_body(e, carry2):
                    ev = jnp.full((16,), e, jnp.int32)
                    p = plsc.load_gather(pos_v, [lv, ev])
                    for k in range(8):
                        v = plsc.load_gather(rows_v.at[b], [blk_rows[k], ev])
                        cbuf_v[b, e >> 3, e & 7, pl.ds(16 * k, 16)] = v + p
                    return carry2

                lax.fori_loop(0, E, e_body, 0)
                pltpu.async_copy(cbuf_v.at[b],
                                 out_hbm.at[lpos, pl.ds(0, eh), bh], osems[b])
            return carry

        lax.fori_loop(0, per_w // _NBUF, group_body, 0)
        for b in range(_NBUF):
            wait_out(b)

    return run


def kernel(src, seg, word_table, pos_table):
    B, L = src.shape
    V, E = word_table.shape
    srcT = jnp.transpose(src).reshape(L * B // _BB, _BB).astype(jnp.int32)
    run = _build(B, L, E)
    out5 = run(srcT, word_table, pos_table)
    out = out5.transpose(2, 4, 0, 1, 3).reshape(B, L, E)
    return out


# speedup vs baseline: 1.7214x; 1.7214x over previous
import functools

import jax
import jax.numpy as jnp
from jax import lax
from jax.experimental import pallas as pl
from jax.experimental.pallas import tpu as pltpu
from jax.experimental.pallas import tpu_sc as plsc

_INFO = plsc.get_sparse_core_info()
_NC, _NS, _LANES = _INFO.num_cores, _INFO.num_subcores, _INFO.num_lanes
_NW = _NC * _NS
_BB = 128
_NBUF = 4


def _build(B, L, E):
    assert E % 8 == 0 and B % _BB == 0
    eh = E // 8
    nbh = B // _BB
    units = L * nbh
    assert units % _NW == 0
    per_w = units // _NW
    assert per_w % _NBUF == 0

    mesh = plsc.VectorSubcoreMesh(core_axis_name="c", subcore_axis_name="s")

    @functools.partial(
        pl.kernel,
        mesh=mesh,
        out_type=jax.ShapeDtypeStruct((L, eh, nbh, 8 * _BB), jnp.float32),
        compiler_params=pltpu.CompilerParams(use_tc_tiling_on_sc=False,
                                             needs_layout_passes=False),
        scratch_types=(
            [
                pltpu.VMEM((per_w, _BB), jnp.int32),
                pltpu.VMEM((L, E), jnp.float32),
                pltpu.VMEM((_NBUF, _BB, E), jnp.float32),
                pltpu.VMEM((_NBUF, eh, 8 * _BB), jnp.float32),
            ]
            + [pltpu.SemaphoreType.DMA] * (2 * _NBUF)
        ),
    )
    def run(srcT_hbm, word_hbm, pos_hbm, out_hbm, idx_v, pos_v, rows_v,
            cbuf_v, *sems):
        gsems, osems = sems[:_NBUF], sems[_NBUF:]
        wid = lax.axis_index("s") * _NC + lax.axis_index("c")
        u0 = wid * per_w
        pltpu.sync_copy(srcT_hbm.at[pl.ds(u0, per_w)], idx_v)
        pltpu.sync_copy(pos_hbm.at[pl.ds(0, L)], pos_v)

        lane = lax.iota(jnp.int32, 16)
        rot = [(lane + t) & 15 for t in range(16)]
        rot_eh = [r >> 3 for r in rot]
        rot_elb = [(r & 7) * _BB for r in rot]

        def issue_gather(i, b):
            pltpu.async_copy(word_hbm.at[idx_v.at[i]], rows_v.at[b], gsems[b])

        def wait_gather(b):
            pltpu.make_async_copy(word_hbm.at[idx_v.at[0]], rows_v.at[b],
                                  gsems[b]).wait()

        def wait_out(b):
            pltpu.make_async_copy(cbuf_v.at[b],
                                  out_hbm.at[0, pl.ds(0, eh), 0], osems[b]).wait()

        issue_gather(0, 0)
        issue_gather(1, 1)

        def group_body(g, carry):
            for b in range(_NBUF):
                i = g * _NBUF + b
                inext = i + 2
                bnext = (b + 2) % _NBUF

                @pl.when(inext < per_w)
                def _prefetch():
                    issue_gather(inext, bnext)

                wait_gather(b)

                @pl.when(i >= _NBUF)
                def _drain():
                    wait_out(b)

                u = u0 + i
                lpos = u >> 5
                bh = u & (nbh - 1)

                lv = jnp.full((16,), lpos, jnp.int32)

                def eb_body(ebi, carry2):
                    e0 = ebi << 4
                    ev = [rot[t] + e0 for t in range(16)]
                    prot = [plsc.load_gather(pos_v, [lv, ev[t]])
                            for t in range(16)]
                    eh0 = ebi << 1

                    def bb_body(bb, carry3):
                        bvec = lane + (bb << 4)
                        for t in range(16):
                            v = plsc.load_gather(rows_v.at[b], [bvec, ev[t]])
                            plsc.store_scatter(
                                cbuf_v.at[b],
                                [rot_eh[t] + eh0, rot_elb[t] + bvec],
                                v + prot[t])
                        return carry3

                    lax.fori_loop(0, _BB // 16, bb_body, 0)
                    return carry2

                lax.fori_loop(0, E // 16, eb_body, 0)
                pltpu.async_copy(cbuf_v.at[b],
                                 out_hbm.at[lpos, pl.ds(0, eh), bh], osems[b])
            return carry

        lax.fori_loop(0, per_w // _NBUF, group_body, 0)
        for b in range(_NBUF):
            wait_out(b)

    return run


def kernel(src, seg, word_table, pos_table):
    B, L = src.shape
    V, E = word_table.shape
    srcT = jnp.transpose(src).reshape(L * B // _BB, _BB).astype(jnp.int32)
    run = _build(B, L, E)
    out5 = run(srcT, word_table, pos_table).reshape(L, E // 8, B // _BB, 8, _BB)
    out = out5.transpose(2, 4, 0, 1, 3).reshape(B, L, E)
    return out
